# Initial kernel scaffold; baseline (speedup 1.0000x reference)
import jax
import jax.numpy as jnp
from jax.experimental import pallas as pl


def kernel(image, index):
    raise NotImplementedError("write your pallas kernel here")


# speedup vs baseline: 8.5326x; 8.5326x over previous
import jax
import jax.numpy as jnp
from jax import lax
from jax.experimental import pallas as pl
from jax.experimental.pallas import tpu as pltpu

_L = 256
_H = _W = 256
_R = 16
_NCHUNK = _H // _R


def _glcm_var_kernel(band_ref, out_ref):
    band = band_ref[...]
    lo = jnp.min(band)
    hi = jnp.max(band)
    scaled = (band - lo) / jnp.maximum(hi - lo, jnp.float32(1e-12))
    codes = jnp.clip(jnp.round(scaled * 255.0), 0.0, 255.0)
    u8 = codes.astype(jnp.bfloat16)

    neg_col = jnp.full((_H, 1), -1.0, jnp.bfloat16)
    neg_row = jnp.full((1, _W), -1.0, jnp.bfloat16)
    down = jnp.concatenate([u8[1:, :], neg_row], axis=0)
    b01 = jnp.concatenate([u8[:, 1:], neg_col], axis=1)
    b11 = jnp.concatenate([down[:, 1:], neg_col], axis=1)
    b10 = down
    b1m = jnp.concatenate([neg_col, down[:, :-1]], axis=1)

    iota_lvl = lax.broadcasted_iota(jnp.int32, (_R, _W, _L), 2).astype(jnp.bfloat16)
    one = jnp.bfloat16(1.0)
    zero = jnp.bfloat16(0.0)

    def onehot(x):
        m = x[:, :, None] == iota_lvl
        return jnp.where(m, one, zero).reshape(_R * _W, _L)

    acc = jnp.zeros((_L, 4 * _L), jnp.float32)
    for ci in range(_NCHUNK):
        sl = slice(ci * _R, (ci + 1) * _R)
        oh_a = onehot(u8[sl])
        oh_b = jnp.concatenate(
            [onehot(b01[sl]), onehot(b11[sl]), onehot(b10[sl]), onehot(b1m[sl])],
            axis=1,
        )
        acc = acc + lax.dot_general(
            oh_a, oh_b, (((0,), (0,)), ((), ())),
            preferred_element_type=jnp.float32,
        )

    c0 = acc[:, 0 * _L:1 * _L]
    c1 = acc[:, 1 * _L:2 * _L]
    c2 = acc[:, 2 * _L:3 * _L]
    c3 = acc[:, 3 * _L:4 * _L]
    mean = (c0 + c1 + c2 + c3) * 0.25
    d0 = c0 - mean
    d1 = c1 - mean
    d2 = c2 - mean
    d3 = c3 - mean
    out_ref[...] = (d0 * d0 + d1 * d1 + d2 * d2 + d3 * d3) * 0.25


def _glcm_var(band, interpret=False):
    return pl.pallas_call(
        _glcm_var_kernel,
        out_shape=jax.ShapeDtypeStruct((_H, _W), jnp.float32),
        compiler_params=pltpu.CompilerParams(
            vmem_limit_bytes=56 * 1024 * 1024,
        ),
        name="glcm_var",
        interpret=interpret,
    )(band)


def kernel(image, index):
    band = lax.dynamic_index_in_dim(image, index, axis=0, keepdims=False)
    var = _glcm_var(band)
    return jnp.concatenate([image, var[None]], axis=0)


# speedup vs baseline: 12.1332x; 1.4220x over previous
import jax
import jax.numpy as jnp
from jax import lax
from jax.experimental import pallas as pl
from jax.experimental.pallas import tpu as pltpu

_L = 256
_H = _W = 256
_NB = 180
_R = 16
_NSTEP = _H // _R
_BPS = 12


def _quantize_and_shift(band):
    lo = jnp.min(band)
    hi = jnp.max(band)
    scaled = (band - lo) / jnp.maximum(hi - lo, jnp.float32(1e-12))
    codes = jnp.clip(jnp.round(scaled * 255.0), 0.0, 255.0)
    u8 = codes.astype(jnp.bfloat16)

    neg_col = jnp.full((_H, 1), -1.0, jnp.bfloat16)
    neg_row = jnp.full((1, _W), -1.0, jnp.bfloat16)
    down = jnp.concatenate([u8[1:, :], neg_row], axis=0)
    b01 = jnp.concatenate([u8[:, 1:], neg_col], axis=1)
    b11 = jnp.concatenate([down[:, 1:], neg_col], axis=1)
    b10 = down
    b1m = jnp.concatenate([neg_col, down[:, :-1]], axis=1)
    return u8, b01, b11, b10, b1m


def _onehot(x, iota_lvl):
    m = x[:, :, None] == iota_lvl
    return jnp.where(m, jnp.bfloat16(1.0), jnp.bfloat16(0.0)).reshape(_R * _W, _L)


def _fused_kernel(idx_ref, band_ref, img_ref, out_ref, sc_ref, acc_ref):
    del idx_ref
    j = pl.program_id(0)

    @pl.when(j < _NSTEP - 1)
    def _copy():
        out_ref[...] = img_ref[...]

    @pl.when(j == 0)
    def _init():
        u8, b01, b11, b10, b1m = _quantize_and_shift(band_ref[0])
        sc_ref[0] = u8
        sc_ref[1] = b01
        sc_ref[2] = b11
        sc_ref[3] = b10
        sc_ref[4] = b1m
        acc_ref[...] = jnp.zeros_like(acc_ref)

    iota_lvl = lax.broadcasted_iota(jnp.int32, (_R, _W, _L), 2).astype(jnp.bfloat16)
    off = pl.multiple_of(j * _R, _R)
    oh_a = _onehot(sc_ref[0, pl.ds(off, _R), :], iota_lvl)
    oh_b = jnp.concatenate(
        [_onehot(sc_ref[k, pl.ds(off, _R), :], iota_lvl) for k in (1, 2, 3, 4)],
        axis=1,
    )
    acc_ref[...] += lax.dot_general(
        oh_a, oh_b, (((0,), (0,)), ((), ())),
        preferred_element_type=jnp.float32,
    )

    @pl.when(j == _NSTEP - 1)
    def _finish():
        acc = acc_ref[...]
        c0 = acc[:, 0 * _L:1 * _L]
        c1 = acc[:, 1 * _L:2 * _L]
        c2 = acc[:, 2 * _L:3 * _L]
        c3 = acc[:, 3 * _L:4 * _L]
        mean = (c0 + c1 + c2 + c3) * 0.25
        d0 = c0 - mean
        d1 = c1 - mean
        d2 = c2 - mean
        d3 = c3 - mean
        out_ref[0] = (d0 * d0 + d1 * d1 + d2 * d2 + d3 * d3) * 0.25


def kernel(image, index):
    idx = jnp.asarray(index, jnp.int32).reshape(1)
    return pl.pallas_call(
        _fused_kernel,
        out_shape=jax.ShapeDtypeStruct((_NB + 1, _H, _W), jnp.float32),
        grid_spec=pltpu.PrefetchScalarGridSpec(
            num_scalar_prefetch=1,
            grid=(_NSTEP,),
            in_specs=[
                pl.BlockSpec((1, _H, _W), lambda j, i: (i[0], 0, 0)),
                pl.BlockSpec((_BPS, _H, _W),
                             lambda j, i: (j - j // (_NSTEP - 1), 0, 0)),
            ],
            out_specs=pl.BlockSpec((_BPS, _H, _W), lambda j, i: (j, 0, 0)),
            scratch_shapes=[
                pltpu.VMEM((5, _H, _W), jnp.bfloat16),
                pltpu.VMEM((_L, 4 * _L), jnp.float32),
            ],
        ),
        compiler_params=pltpu.CompilerParams(
            dimension_semantics=("arbitrary",),
            vmem_limit_bytes=56 * 1024 * 1024,
        ),
        name="glcm_append_fused",
    )(idx, image, image)


# speedup vs baseline: 16.5556x; 1.3645x over previous
import jax
import jax.numpy as jnp
from jax import lax
from jax.experimental import pallas as pl
from jax.experimental.pallas import tpu as pltpu

_L = 256
_H = _W = 256
_NB = 180
_R = 16
_NSTEP = _H // _R
_BPS = 12


def _quantize_and_shift(band):
    lo = jnp.min(band)
    hi = jnp.max(band)
    scaled = (band - lo) / jnp.maximum(hi - lo, jnp.float32(1e-12))
    codes = jnp.clip(jnp.round(scaled * 255.0), 0.0, 255.0)
    u8 = codes.astype(jnp.bfloat16)

    neg_col = jnp.full((_H, 1), -1.0, jnp.bfloat16)
    neg_row = jnp.full((1, _W), -1.0, jnp.bfloat16)
    down = jnp.concatenate([u8[1:, :], neg_row], axis=0)
    b01 = jnp.concatenate([u8[:, 1:], neg_col], axis=1)
    b11 = jnp.concatenate([down[:, 1:], neg_col], axis=1)
    b10 = down
    b1m = jnp.concatenate([neg_col, down[:, :-1]], axis=1)
    return u8, b01, b11, b10, b1m


_F8 = jnp.float8_e4m3fn


def _onehot(x, iota_lvl):
    m = x[:, :, None] == iota_lvl
    oh = jnp.where(m, jnp.bfloat16(1.0), jnp.bfloat16(0.0))
    return oh.astype(_F8).reshape(_R * _W, _L)


def _fused_kernel(idx_ref, band_ref, img_ref, out_ref, sc_ref, acc_ref):
    del idx_ref
    j = pl.program_id(0)

    @pl.when(j < _NSTEP - 1)
    def _copy():
        out_ref[...] = img_ref[...]

    @pl.when(j == 0)
    def _init():
        u8, b01, b11, b10, b1m = _quantize_and_shift(band_ref[0])
        sc_ref[0] = u8
        sc_ref[1] = b01
        sc_ref[2] = b11
        sc_ref[3] = b10
        sc_ref[4] = b1m
        acc_ref[...] = jnp.zeros_like(acc_ref)

    iota_lvl = lax.broadcasted_iota(jnp.int32, (_R, _W, _L), 2).astype(jnp.bfloat16)
    off = pl.multiple_of(j * _R, _R)
    oh_a = _onehot(sc_ref[0, pl.ds(off, _R), :], iota_lvl)
    oh_b = jnp.concatenate(
        [_onehot(sc_ref[k, pl.ds(off, _R), :], iota_lvl) for k in (1, 2, 3, 4)],
        axis=1,
    )
    acc_ref[...] += lax.dot_general(
        oh_a, oh_b, (((0,), (0,)), ((), ())),
        preferred_element_type=jnp.float32,
    )

    @pl.when(j == _NSTEP - 1)
    def _finish():
        acc = acc_ref[...]
        c0 = acc[:, 0 * _L:1 * _L]
        c1 = acc[:, 1 * _L:2 * _L]
        c2 = acc[:, 2 * _L:3 * _L]
        c3 = acc[:, 3 * _L:4 * _L]
        mean = (c0 + c1 + c2 + c3) * 0.25
        d0 = c0 - mean
        d1 = c1 - mean
        d2 = c2 - mean
        d3 = c3 - mean
        out_ref[0] = (d0 * d0 + d1 * d1 + d2 * d2 + d3 * d3) * 0.25


def kernel(image, index):
    idx = jnp.asarray(index, jnp.int32).reshape(1)
    return pl.pallas_call(
        _fused_kernel,
        out_shape=jax.ShapeDtypeStruct((_NB + 1, _H, _W), jnp.float32),
        grid_spec=pltpu.PrefetchScalarGridSpec(
            num_scalar_prefetch=1,
            grid=(_NSTEP,),
            in_specs=[
                pl.BlockSpec((1, _H, _W), lambda j, i: (i[0], 0, 0)),
                pl.BlockSpec((_BPS, _H, _W),
                             lambda j, i: (j - j // (_NSTEP - 1), 0, 0)),
            ],
            out_specs=pl.BlockSpec((_BPS, _H, _W), lambda j, i: (j, 0, 0)),
            scratch_shapes=[
                pltpu.VMEM((5, _H, _W), jnp.bfloat16),
                pltpu.VMEM((_L, 4 * _L), jnp.float32),
            ],
        ),
        compiler_params=pltpu.CompilerParams(
            dimension_semantics=("arbitrary",),
            vmem_limit_bytes=56 * 1024 * 1024,
        ),
        name="glcm_append_fused",
    )(idx, image, image)
